# baseline (speedup 1.0000x reference)
import functools

import jax
import jax.numpy as jnp
from jax import lax
from jax.experimental import pallas as pl
from jax.experimental.pallas import tpu as pltpu
from jax.experimental.pallas import tpu_sc as plsc

K = 512
D = 32
BETA = 0.25
BG = 8

NC = 2
NS = 16
NW = NC * NS


def _argmin_kernel(z_ref, cb_ref, idx_ref, loss_ref):
    i = pl.program_id(0)
    ng = pl.num_programs(0)
    bn = BG * z_ref.shape[1]
    z = z_ref[...].reshape(bn, D)
    cb = cb_ref[...]
    z2 = jnp.sum(z * z, axis=1, keepdims=True)
    e2 = jnp.sum(cb * cb, axis=1)[None, :]
    cross = jax.lax.dot_general(
        z, cb, (((1,), (1,)), ((), ())),
        preferred_element_type=jnp.float32,
        precision=jax.lax.Precision.DEFAULT)
    dist = z2 - 2.0 * cross + e2
    minv = jnp.min(dist, axis=1, keepdims=True)
    iota_row = jax.lax.broadcasted_iota(
        jnp.int32, (1, K), 1).astype(jnp.float32) - 256.0
    idxs = jnp.min(jnp.where(dist <= minv, iota_row, 256.0),
                   axis=1, keepdims=True)
    idx_bc = jax.lax.dot_general(
        idxs.astype(jnp.bfloat16), jnp.ones((1, 128), jnp.bfloat16),
        (((1,), (0,)), ((), ())),
        preferred_element_type=jnp.float32)
    idx_t = jnp.swapaxes(idx_bc, 0, 1)
    idx_ref[...] = (idx_t[0:1, :] + 256.0).astype(jnp.int32)[None]
    psum = jnp.sum(minv)

    @pl.when(i == 0)
    def _init():
        loss_ref[0, 0] = 0.0

    loss_ref[0, 0] += psum

    @pl.when(i == ng - 1)
    def _finish():
        scale = (1.0 + BETA) / jnp.float32(ng * bn * D)
        loss_ref[0, 0] = loss_ref[0, 0] * scale


CHUNK = 256


def _gather_body(table_hbm, idx_hbm, out_hbm, idx_v, rows_v, sem):
    wid = lax.axis_index("s") * NC + lax.axis_index("c")
    base = wid * (CHUNK * 4)
    for c in range(4):
        off = base + c * CHUNK
        pltpu.sync_copy(idx_hbm.at[pl.ds(off, CHUNK)], idx_v)
        pltpu.async_copy(table_hbm.at[idx_v], rows_v, sem).wait()
        pltpu.sync_copy(rows_v, out_hbm.at[pl.ds(off, CHUNK)])


def kernel(z, codebook):
    g, s, _ = z.shape
    n = g * s
    bn = BG * s
    grid = g // BG
    idx3d, loss = pl.pallas_call(
        _argmin_kernel,
        grid=(grid,),
        in_specs=[
            pl.BlockSpec((BG, s, D), lambda i: (i, 0, 0)),
            pl.BlockSpec((K, D), lambda i: (0, 0)),
        ],
        out_specs=[
            pl.BlockSpec((1, 1, bn), lambda i: (i, 0, 0)),
            pl.BlockSpec(memory_space=pltpu.SMEM),
        ],
        out_shape=[
            jax.ShapeDtypeStruct((grid, 1, bn), jnp.int32),
            jax.ShapeDtypeStruct((1, 1), jnp.float32),
        ],
    )(z, codebook)
    idx_flat = idx3d.reshape(n)
    b_per_w = n // NW
    cb_pad = jnp.pad(codebook, ((0, 0), (0, 128 - D)))
    mesh = plsc.VectorSubcoreMesh(core_axis_name="c", subcore_axis_name="s")
    out128 = pl.kernel(
        _gather_body,
        out_type=jax.ShapeDtypeStruct((n, 128), jnp.float32),
        mesh=mesh,
        scratch_types=[
            pltpu.VMEM((CHUNK,), jnp.int32),
            pltpu.VMEM((CHUNK, 128), jnp.float32),
            pltpu.SemaphoreType.DMA,
        ],
    )(cb_pad, idx_flat)
    return out128[:, :D].reshape(z.shape), loss[0, 0]


# speedup vs baseline: 1.4920x; 1.4920x over previous
import jax
import jax.numpy as jnp
from jax.experimental import pallas as pl
from jax.experimental.pallas import tpu as pltpu

K = 512
D = 32
BETA = 0.25
BG = 8


def _vq_kernel(z_ref, cb_ref, out_ref, loss_ref, e2_ref):
    i = pl.program_id(0)
    ng = pl.num_programs(0)
    bn = BG * z_ref.shape[1]
    z = z_ref[...].reshape(bn, D)
    cb = cb_ref[...]
    z2 = jnp.sum(z * z, axis=1, keepdims=True)

    @pl.when(i == 0)
    def _precompute():
        e2_ref[...] = jnp.sum(cb * cb, axis=1)[None, :]

    e2 = e2_ref[...]
    cross2 = jax.lax.dot_general(
        z, cb + cb, (((1,), (1,)), ((), ())),
        preferred_element_type=jnp.float32,
        precision=jax.lax.Precision.DEFAULT)
    dist = z2 - cross2 + e2
    minv = jnp.min(dist, axis=1, keepdims=True)
    iota_row = jax.lax.broadcasted_iota(
        jnp.int32, (1, K), 1).astype(jnp.float32) - 256.0
    idxs = jnp.min(jnp.where(dist <= minv, iota_row, 256.0),
                   axis=1, keepdims=True)
    idx_bcast = jax.lax.dot_general(
        idxs.astype(jnp.bfloat16), jnp.ones((1, K), jnp.bfloat16),
        (((1,), (0,)), ((), ())),
        preferred_element_type=jnp.float32)
    onehot = jnp.where(iota_row == idx_bcast,
                       1.0, 0.0).astype(jnp.bfloat16)
    q = jax.lax.dot_general(
        onehot, cb.astype(jnp.bfloat16), (((1,), (0,)), ((), ())),
        preferred_element_type=jnp.float32)
    out_ref[...] = q.reshape(out_ref.shape)
    psum = jnp.sum(minv)

    @pl.when(i == 0)
    def _init():
        loss_ref[0, 0] = 0.0

    loss_ref[0, 0] += psum

    @pl.when(i == ng - 1)
    def _finish():
        scale = (1.0 + BETA) / jnp.float32(ng * bn * D)
        loss_ref[0, 0] = loss_ref[0, 0] * scale


def kernel(z, codebook):
    g, s, _ = z.shape
    out, loss = pl.pallas_call(
        _vq_kernel,
        grid=(g // BG,),
        in_specs=[
            pl.BlockSpec((BG, s, D), lambda i: (i, 0, 0)),
            pl.BlockSpec((K, D), lambda i: (0, 0)),
        ],
        out_specs=[
            pl.BlockSpec((BG, s, D), lambda i: (i, 0, 0)),
            pl.BlockSpec(memory_space=pltpu.SMEM),
        ],
        out_shape=[
            jax.ShapeDtypeStruct(z.shape, jnp.float32),
            jax.ShapeDtypeStruct((1, 1), jnp.float32),
        ],
        scratch_shapes=[pltpu.VMEM((1, K), jnp.float32)],
    )(z, codebook)
    return out, loss[0, 0]


# speedup vs baseline: 1.5869x; 1.0636x over previous
import jax
import jax.numpy as jnp
from jax.experimental import pallas as pl
from jax.experimental.pallas import tpu as pltpu

K = 512
D = 32
BETA = 0.25
BG = 8


def _vq_kernel(z_ref, cb_ref, out_ref, loss_ref):
    i = pl.program_id(0)
    ng = pl.num_programs(0)
    bn = BG * z_ref.shape[1]
    z = z_ref[...].reshape(bn, D)
    cb = cb_ref[...]
    z2 = jnp.sum(z * z, axis=1, keepdims=True)
    e2 = jnp.sum(cb * cb, axis=1)[None, :]
    cross = jax.lax.dot_general(
        z, cb, (((1,), (1,)), ((), ())),
        preferred_element_type=jnp.float32,
        precision=jax.lax.Precision.DEFAULT)
    dist = z2 - 2.0 * cross + e2
    minv = jnp.min(dist, axis=1, keepdims=True)
    iota_row = jax.lax.broadcasted_iota(
        jnp.int32, (1, K), 1).astype(jnp.float32) - 256.0
    idxs = jnp.min(jnp.where(dist <= minv, iota_row, 256.0),
                   axis=1, keepdims=True)
    idx_bcast = jax.lax.dot_general(
        idxs.astype(jnp.bfloat16), jnp.ones((1, K), jnp.bfloat16),
        (((1,), (0,)), ((), ())),
        preferred_element_type=jnp.float32)
    onehot = jnp.where(iota_row == idx_bcast,
                       1.0, 0.0).astype(jnp.bfloat16)
    q = jax.lax.dot_general(
        onehot, cb.astype(jnp.bfloat16), (((1,), (0,)), ((), ())),
        preferred_element_type=jnp.float32)
    out_ref[...] = q.reshape(out_ref.shape)
    psum = jnp.sum(minv)

    @pl.when(i == 0)
    def _init():
        loss_ref[0, 0] = 0.0

    loss_ref[0, 0] += psum

    @pl.when(i == ng - 1)
    def _finish():
        scale = (1.0 + BETA) / jnp.float32(ng * bn * D)
        loss_ref[0, 0] = loss_ref[0, 0] * scale


def kernel(z, codebook):
    g, s, _ = z.shape
    out, loss = pl.pallas_call(
        _vq_kernel,
        grid=(g // BG,),
        in_specs=[
            pl.BlockSpec((BG, s, D), lambda i: (i, 0, 0)),
            pl.BlockSpec((K, D), lambda i: (0, 0)),
        ],
        out_specs=[
            pl.BlockSpec((BG, s, D), lambda i: (i, 0, 0)),
            pl.BlockSpec(memory_space=pltpu.SMEM),
        ],
        out_shape=[
            jax.ShapeDtypeStruct(z.shape, jnp.float32),
            jax.ShapeDtypeStruct((1, 1), jnp.float32),
        ],
    )(z, codebook)
    return out, loss[0, 0]


# speedup vs baseline: 1.5986x; 1.0074x over previous
import jax
import jax.numpy as jnp
from jax.experimental import pallas as pl
from jax.experimental.pallas import tpu as pltpu

K = 512
D = 32
BETA = 0.25
BG = 4


def _vq_kernel(z_ref, cb_ref, out_ref, loss_ref):
    i = pl.program_id(0)
    ng = pl.num_programs(0)
    bn = BG * z_ref.shape[1]
    z = z_ref[...].reshape(bn, D)
    cb = cb_ref[...]
    z2 = jnp.sum(z * z, axis=1, keepdims=True)
    e2 = jnp.sum(cb * cb, axis=1)[None, :]
    cross = jax.lax.dot_general(
        z, cb, (((1,), (1,)), ((), ())),
        preferred_element_type=jnp.float32,
        precision=jax.lax.Precision.DEFAULT)
    dist = z2 - 2.0 * cross + e2
    minv = jnp.min(dist, axis=1, keepdims=True)
    iota_row = jax.lax.broadcasted_iota(
        jnp.int32, (1, K), 1).astype(jnp.float32) - 256.0
    idxs = jnp.min(jnp.where(dist <= minv, iota_row, 256.0),
                   axis=1, keepdims=True)
    idx_bcast = jax.lax.dot_general(
        idxs.astype(jnp.bfloat16), jnp.ones((1, K), jnp.bfloat16),
        (((1,), (0,)), ((), ())),
        preferred_element_type=jnp.float32)
    onehot = jnp.where(iota_row == idx_bcast,
                       1.0, 0.0).astype(jnp.bfloat16)
    q = jax.lax.dot_general(
        onehot, cb.astype(jnp.bfloat16), (((1,), (0,)), ((), ())),
        preferred_element_type=jnp.float32)
    out_ref[...] = q.reshape(out_ref.shape)
    psum = jnp.sum(minv)

    @pl.when(i == 0)
    def _init():
        loss_ref[0, 0] = 0.0

    loss_ref[0, 0] += psum

    @pl.when(i == ng - 1)
    def _finish():
        scale = (1.0 + BETA) / jnp.float32(ng * bn * D)
        loss_ref[0, 0] = loss_ref[0, 0] * scale


def kernel(z, codebook):
    g, s, _ = z.shape
    out, loss = pl.pallas_call(
        _vq_kernel,
        grid=(g // BG,),
        in_specs=[
            pl.BlockSpec((BG, s, D), lambda i: (i, 0, 0)),
            pl.BlockSpec((K, D), lambda i: (0, 0)),
        ],
        out_specs=[
            pl.BlockSpec((BG, s, D), lambda i: (i, 0, 0)),
            pl.BlockSpec(memory_space=pltpu.SMEM),
        ],
        out_shape=[
            jax.ShapeDtypeStruct(z.shape, jnp.float32),
            jax.ShapeDtypeStruct((1, 1), jnp.float32),
        ],
    )(z, codebook)
    return out, loss[0, 0]
